# baseline (speedup 1.0000x reference)
import jax
import jax.numpy as jnp
from jax import lax
from jax.experimental import pallas as pl
from jax.experimental.pallas import tpu as pltpu
from jax.experimental.pallas import tpu_sc as plsc

H, W, C = 512, 512, 63
NCORES, NSUB = 2, 16
NW = NCORES * NSUB
RPW = H // NW
LANES = 16
JCH = W // LANES


def _sc_body(img_hbm, out_hbm, imgv, buf0, buf1, buf2, sem0, sem1, sem2):
    wid = lax.axis_index("s") * NCORES + lax.axis_index("c")
    r0 = wid * RPW
    bufs = (buf0, buf1, buf2)
    sems = (sem0, sem1, sem2)

    pltpu.sync_copy(img_hbm.at[0, pl.ds(r0, RPW), :], imgv)

    def compute_unit(c, buf):
        @pl.loop(0, RPW)
        def _(i):
            for jc in range(JCH):
                vch = imgv[i, pl.ds(jc * LANES, LANES)]
                m = vch == (c + 1)
                buf[i, pl.ds(jc * LANES, LANES)] = m.astype(jnp.int32)

    def start_unit(c, b):
        compute_unit(c, bufs[b])
        pltpu.async_copy(bufs[b], out_hbm.at[c, pl.ds(r0, RPW)], sems[b])

    def wait_unit(c, b):
        pltpu.make_async_copy(
            bufs[b], out_hbm.at[c, pl.ds(r0, RPW)], sems[b]).wait()

    for b in range(3):
        start_unit(jnp.int32(b), b)

    @pl.loop(3, C - 3, step=3)
    def _(c):
        for b in range(3):
            wait_unit(c + b - 3, b)
            start_unit(c + b, b)

    for b in range(3):
        wait_unit(jnp.int32(C - 6 + b), b)
        start_unit(jnp.int32(C - 3 + b), b)
    for b in range(3):
        wait_unit(jnp.int32(C - 3 + b), b)


@jax.jit
def _onehot(img):
    run = pl.kernel(
        _sc_body,
        out_type=jax.ShapeDtypeStruct((C, H, W), jnp.int32),
        mesh=plsc.VectorSubcoreMesh(core_axis_name="c", subcore_axis_name="s"),
        compiler_params=pltpu.CompilerParams(
            needs_layout_passes=False, use_tc_tiling_on_sc=True),
        scratch_types=[
            pltpu.VMEM((RPW, W), jnp.int32),
            pltpu.VMEM((RPW, W), jnp.int32),
            pltpu.VMEM((RPW, W), jnp.int32),
            pltpu.VMEM((RPW, W), jnp.int32),
            pltpu.SemaphoreType.DMA,
            pltpu.SemaphoreType.DMA,
            pltpu.SemaphoreType.DMA,
        ],
    )
    return run(img)


def kernel(img):
    return _onehot(img).transpose(1, 2, 0)


# speedup vs baseline: 1.0027x; 1.0027x over previous
import jax
import jax.numpy as jnp
from jax import lax
from jax.experimental import pallas as pl
from jax.experimental.pallas import tpu as pltpu
from jax.experimental.pallas import tpu_sc as plsc

H, W, C = 512, 512, 63
NCORES, NSUB = 2, 16
NW = NCORES * NSUB
RPW = H // NW
LANES = 16
JCH = W // LANES


def _sc_body(img_hbm, out_hbm, imgv, buf0, buf1, sem0, sem1):
    wid = lax.axis_index("s") * NCORES + lax.axis_index("c")
    r0 = wid * RPW
    bufs = (buf0, buf1)
    sems = (sem0, sem1)

    pltpu.sync_copy(img_hbm.at[0, pl.ds(r0, RPW), :], imgv)

    def compute_unit(c, buf):
        @pl.loop(0, RPW)
        def _(i):
            for jc in range(JCH):
                vch = imgv[i, pl.ds(jc * LANES, LANES)]
                m = vch == (c + 1)
                buf[i, pl.ds(jc * LANES, LANES)] = m.astype(jnp.int32)

    def start_unit(c, b):
        compute_unit(c, bufs[b])
        pltpu.async_copy(bufs[b], out_hbm.at[c, pl.ds(r0, RPW)], sems[b])

    def wait_unit(c, b):
        pltpu.make_async_copy(
            bufs[b], out_hbm.at[c, pl.ds(r0, RPW)], sems[b]).wait()

    for b in range(2):
        start_unit(jnp.int32(b), b)

    @pl.loop(2, C - 1, step=2)
    def _(c):
        for b in range(2):
            wait_unit(c + b - 2, b)
            start_unit(c + b, b)

    wait_unit(jnp.int32(C - 3), 0)
    start_unit(jnp.int32(C - 1), 0)
    wait_unit(jnp.int32(C - 2), 1)
    wait_unit(jnp.int32(C - 1), 0)


@jax.jit
def _onehot(img):
    run = pl.kernel(
        _sc_body,
        out_type=jax.ShapeDtypeStruct((C, H, W), jnp.int32),
        mesh=plsc.VectorSubcoreMesh(core_axis_name="c", subcore_axis_name="s"),
        compiler_params=pltpu.CompilerParams(
            needs_layout_passes=False, use_tc_tiling_on_sc=True),
        scratch_types=[
            pltpu.VMEM((RPW, W), jnp.int32),
            pltpu.VMEM((RPW, W), jnp.int32),
            pltpu.VMEM((RPW, W), jnp.int32),
            pltpu.SemaphoreType.DMA,
            pltpu.SemaphoreType.DMA,
        ],
    )
    return run(img)


def kernel(img):
    return _onehot(img).transpose(1, 2, 0)


# speedup vs baseline: 1.0105x; 1.0078x over previous
import jax
import jax.numpy as jnp
from jax import lax
from jax.experimental import pallas as pl
from jax.experimental.pallas import tpu as pltpu
from jax.experimental.pallas import tpu_sc as plsc

H, W, C = 512, 512, 63
NCORES, NSUB = 2, 16
NW = NCORES * NSUB
RPW = H // NW
LANES = 16
JCH = W // LANES


def _sc_body(img_hbm, out_hbm, imgv, buf0, buf1, sem0, sem1):
    wid = lax.axis_index("s") * NCORES + lax.axis_index("c")
    r0 = wid * RPW
    bufs = (buf0, buf1)
    sems = (sem0, sem1)

    pltpu.sync_copy(img_hbm.at[0, pl.ds(r0, RPW), :], imgv)

    def compute_unit(c, buf):
        @pl.loop(0, RPW)
        def _(i):
            for jc in range(JCH):
                vch = imgv[i, pl.ds(jc * LANES, LANES)]
                m = vch == (c + 1)
                buf[i, pl.ds(jc * LANES, LANES)] = m.astype(jnp.int32)

    def start_unit(c, b):
        compute_unit(c, bufs[b])
        pltpu.async_copy(bufs[b], out_hbm.at[c, pl.ds(r0, RPW)], sems[b])

    def wait_unit(c, b):
        pltpu.make_async_copy(
            bufs[b], out_hbm.at[c, pl.ds(r0, RPW)], sems[b]).wait()

    for b in range(2):
        start_unit(jnp.int32(b), b)

    @pl.loop(2, C - 1, step=2)
    def _(c):
        for b in range(2):
            wait_unit(c + b - 2, b)
            start_unit(c + b, b)

    wait_unit(jnp.int32(C - 3), 0)
    start_unit(jnp.int32(C - 1), 0)
    wait_unit(jnp.int32(C - 2), 1)
    wait_unit(jnp.int32(C - 1), 0)


@jax.jit
def _onehot(img):
    run = pl.kernel(
        _sc_body,
        out_type=jax.ShapeDtypeStruct((C, H, W), jnp.int32),
        mesh=plsc.VectorSubcoreMesh(core_axis_name="c", subcore_axis_name="s"),
        compiler_params=pltpu.CompilerParams(
            needs_layout_passes=False, use_tc_tiling_on_sc=True,
            internal_scratch_in_bytes=1 << 20),
        scratch_types=[
            pltpu.VMEM((RPW, W), jnp.int32),
            pltpu.VMEM((RPW, W), jnp.int32),
            pltpu.VMEM((RPW, W), jnp.int32),
            pltpu.SemaphoreType.DMA,
            pltpu.SemaphoreType.DMA,
        ],
    )
    return run(img)


def kernel(img):
    return _onehot(img).transpose(1, 2, 0)
